# baseline (speedup 1.0000x reference)
import functools

import jax
import jax.numpy as jnp
from jax import lax
from jax.experimental import pallas as pl
from jax.experimental.pallas import tpu as pltpu
from jax.experimental.pallas import tpu_sc as plsc

HEADS = 4
CH = 16
HID = 64
NUM_BASIS = 16
MAX_RADIUS = 5.0
EDGE_PAD = 32
WIDE = 128

NCORES = 2
NSUB = 16
NW = NCORES * NSUB
LANES = 16
CHUNK = 128
NEG = -1e30

_f32 = jnp.float32


def _mm(a, b, dims):
    return lax.dot_general(a, b, (dims, ((), ())),
                           preferred_element_type=jnp.float32)


def _pad_nodes(n):
    m = 128 * NSUB
    return ((n + m - 1) // m) * m


def _lin_body(relu, x_ref, w_ref, b_ref, o_ref):
    y = _mm(x_ref[...], w_ref[...], ((1,), (0,))) + b_ref[...]
    o_ref[...] = jnp.maximum(y, 0.0) if relu else y


def tc_linear(x, w, b, blk=2048, relu=False):
    n, din = x.shape
    dout = w.shape[1]
    return pl.pallas_call(
        functools.partial(_lin_body, relu),
        grid=(n // blk,),
        in_specs=[pl.BlockSpec((blk, din), lambda i: (i, 0)),
                  pl.BlockSpec((din, dout), lambda i: (0, 0)),
                  pl.BlockSpec((1, dout), lambda i: (0, 0))],
        out_specs=pl.BlockSpec((blk, dout), lambda i: (i, 0)),
        out_shape=jax.ShapeDtypeStruct((n, dout), _f32),
    )(x, w, b.reshape(1, -1))


def _mlp_body(x_ref, w1_ref, b1_ref, w2_ref, b2_ref, o_ref):
    h = jnp.maximum(_mm(x_ref[...], w1_ref[...], ((1,), (0,))) + b1_ref[...], 0.0)
    o_ref[...] = _mm(h, w2_ref[...], ((1,), (0,))) + b2_ref[...]


def tc_mlp(x, w1, b1, w2, b2, blk=2048):
    n, din = x.shape
    dh = w1.shape[1]
    dout = w2.shape[1]
    return pl.pallas_call(
        _mlp_body,
        grid=(n // blk,),
        in_specs=[pl.BlockSpec((blk, din), lambda i: (i, 0)),
                  pl.BlockSpec((din, dh), lambda i: (0, 0)),
                  pl.BlockSpec((1, dh), lambda i: (0, 0)),
                  pl.BlockSpec((dh, dout), lambda i: (0, 0)),
                  pl.BlockSpec((1, dout), lambda i: (0, 0))],
        out_specs=pl.BlockSpec((blk, dout), lambda i: (i, 0)),
        out_shape=jax.ShapeDtypeStruct((n, dout), _f32),
    )(x, w1, b1.reshape(1, -1), w2, b2.reshape(1, -1))


def _proj4_body(x_ref, wq, bq, wk, bk, wv, bv, ws, bs, kv_o, qq_o, s_o):
    x = x_ref[...]
    q = _mm(x, wq[...], ((1,), (0,))) + bq[...]
    k = _mm(x, wk[...], ((1,), (0,))) + bk[...]
    v = _mm(x, wv[...], ((1,), (0,))) + bv[...]
    kv_o[...] = jnp.concatenate([k, v], axis=1)
    qq_o[...] = jnp.concatenate([q, q], axis=1)
    s_o[...] = _mm(x, ws[...], ((1,), (0,))) + bs[...]


def tc_proj4(x, p, blk=2048):
    n, din = x.shape
    wspec = pl.BlockSpec((din, HID), lambda i: (0, 0))
    bspec = pl.BlockSpec((1, HID), lambda i: (0, 0))
    return pl.pallas_call(
        _proj4_body,
        grid=(n // blk,),
        in_specs=[pl.BlockSpec((blk, din), lambda i: (i, 0)),
                  wspec, bspec, wspec, bspec, wspec, bspec, wspec, bspec],
        out_specs=[pl.BlockSpec((blk, WIDE), lambda i: (i, 0)),
                   pl.BlockSpec((blk, WIDE), lambda i: (i, 0)),
                   pl.BlockSpec((blk, HID), lambda i: (i, 0))],
        out_shape=[jax.ShapeDtypeStruct((n, WIDE), _f32),
                   jax.ShapeDtypeStruct((n, WIDE), _f32),
                   jax.ShapeDtypeStruct((n, HID), _f32)],
    )(x, p['Wq'], p['bq'].reshape(1, -1), p['Wk'], p['bk'].reshape(1, -1),
      p['Wv'], p['bv'].reshape(1, -1), p['Ws'], p['bs'].reshape(1, -1))


def _edge_attr_body(sg_ref, dg_ref, o_ref):
    ev = sg_ref[:, :XP3] - dg_ref[:, :XP3]
    len2 = jnp.sum(ev * ev, axis=1, keepdims=True)
    elen = jnp.sqrt(len2)
    t = elen * _f32((NUM_BASIS + 1) / MAX_RADIUS)
    j = lax.broadcasted_iota(jnp.int32, (1, NUM_BASIS), 1).astype(_f32) + 1.0
    diff = t - j
    emb = jnp.exp(-(diff * diff)) * _f32(NUM_BASIS ** 0.5 / 1.12)
    unit = ev[:, :3] / (elen + 1e-12)
    b = sg_ref.shape[0]
    ones = jnp.ones((b, 1), _f32)
    zeros = jnp.zeros((b, EDGE_PAD - NUM_BASIS - 4), _f32)
    o_ref[...] = jnp.concatenate(
        [emb, ones, _f32(3.0 ** 0.5) * unit, zeros], axis=1)


XP3 = 8


def tc_edge_attr(sg, dg, blk=2000):
    e = sg.shape[0]
    return pl.pallas_call(
        _edge_attr_body,
        grid=(e // blk,),
        in_specs=[pl.BlockSpec((blk, 16), lambda i: (i, 0)),
                  pl.BlockSpec((blk, 16), lambda i: (i, 0))],
        out_specs=pl.BlockSpec((blk, EDGE_PAD), lambda i: (i, 0)),
        out_shape=jax.ShapeDtypeStruct((e, EDGE_PAD), _f32),
    )(sg, dg)


def _alpha_body(qq_ref, kv_ref, ep_ref, a_o, vpe_o):
    ep = ep_ref[...]
    qk = qq_ref[:, :HID] * (kv_ref[:, :HID] + ep)
    r = lax.broadcasted_iota(jnp.int32, (HID, HEADS), 0)
    c = lax.broadcasted_iota(jnp.int32, (HID, HEADS), 1)
    hsel = jnp.where(r // CH == c, _f32(1.0 / (CH ** 0.5)), 0.0)
    a_o[...] = _mm(hsel, qk, ((0,), (1,)))
    vpe_o[...] = kv_ref[:, HID:] + ep


def tc_alpha(qq, kv, ep, blk=6400):
    e = qq.shape[0]
    return pl.pallas_call(
        _alpha_body,
        grid=(e // blk,),
        in_specs=[pl.BlockSpec((blk, WIDE), lambda i: (i, 0)),
                  pl.BlockSpec((blk, WIDE), lambda i: (i, 0)),
                  pl.BlockSpec((blk, HID), lambda i: (i, 0))],
        out_specs=[pl.BlockSpec((HEADS, blk), lambda i: (0, i)),
                   pl.BlockSpec((blk, HID), lambda i: (i, 0))],
        out_shape=[jax.ShapeDtypeStruct((HEADS, e), _f32),
                   jax.ShapeDtypeStruct((e, HID), _f32)],
    )(qq, kv, ep)


def _maxreduce_body(p_ref, o_ref):
    m = jnp.max(p_ref[...], axis=0)
    o_ref[...] = jnp.where(m < _f32(-1e29), 0.0, m)


def _sumreduce_body(p_ref, o_ref):
    o_ref[...] = jnp.sum(p_ref[...], axis=0)


def tc_reduce(parts, kind):
    r, h, n = parts.shape
    body = _maxreduce_body if kind == 'max' else _sumreduce_body
    return pl.pallas_call(
        body,
        grid=(1,),
        in_specs=[pl.BlockSpec((r, h, n), lambda i: (0, 0, 0))],
        out_specs=pl.BlockSpec((h, n), lambda i: (0, 0)),
        out_shape=jax.ShapeDtypeStruct((h, n), _f32),
    )(parts)


def _b4():
    r = lax.broadcasted_iota(jnp.int32, (HEADS, HID), 0)
    c = lax.broadcasted_iota(jnp.int32, (HEADS, HID), 1)
    return jnp.where(c // CH == r, _f32(1.0), 0.0)


def _rpass_body(ex_ref, vpe_ref, o_ref):
    exb = _mm(ex_ref[...], _b4(), ((0,), (0,)))
    rfull = exb * vpe_ref[...]
    o_ref[0] = rfull[:, :HID // 2]
    o_ref[1] = rfull[:, HID // 2:]


def tc_rpass(ex, vpe, blk=6400):
    e = vpe.shape[0]
    return pl.pallas_call(
        _rpass_body,
        grid=(e // blk,),
        in_specs=[pl.BlockSpec((HEADS, blk), lambda i: (0, i)),
                  pl.BlockSpec((blk, HID), lambda i: (i, 0))],
        out_specs=pl.BlockSpec((2, blk, HID // 2), lambda i: (0, i, 0)),
        out_shape=jax.ShapeDtypeStruct((2, e, HID // 2), _f32),
    )(ex, vpe)


def _finalize_body(blk, acc_ref, den_ref, s_ref, o_ref):
    i = pl.program_id(0)
    acc = jnp.concatenate([acc_ref[0], acc_ref[1]], axis=1)
    den = den_ref[:, pl.ds(i * blk, blk)]
    db = _mm(den, _b4(), ((0,), (0,)))
    o_ref[...] = jnp.maximum(acc / (db + _f32(1e-16)) + s_ref[...], 0.0)


def tc_finalize(acc, den, s, blk=2048):
    n = s.shape[0]
    np_ = den.shape[1]
    return pl.pallas_call(
        functools.partial(_finalize_body, blk),
        grid=(n // blk,),
        in_specs=[pl.BlockSpec((2, blk, HID // 2), lambda i: (0, i, 0)),
                  pl.BlockSpec((HEADS, np_), lambda i: (0, 0)),
                  pl.BlockSpec((blk, HID), lambda i: (i, 0))],
        out_specs=pl.BlockSpec((blk, HID), lambda i: (i, 0)),
        out_shape=jax.ShapeDtypeStruct((n, HID), _f32),
    )(acc, den, s)


_MESH = dict(core_axis_name="c", subcore_axis_name="s")


def _wid():
    return lax.axis_index("c") * NSUB + lax.axis_index("s")


def _strided_chunks(nchunks, stride, offset, body):
    total = (nchunks - offset + stride - 1) // stride

    def outer(j, carry):
        body(offset + j * stride)
        return carry

    lax.fori_loop(0, total, outer, 0)


def sc_gather2(ta, tb, src, dst, width=WIDE, tiled=True, sub=2, bw_b=None):
    e = src.shape[0]
    mch = CHUNK * sub
    nchunks = e // mch
    wb = width if bw_b is None else bw_b
    mesh = plsc.VectorSubcoreMesh(**_MESH)
    osh = jax.ShapeDtypeStruct((e, width), _f32)
    oshb = jax.ShapeDtypeStruct((e, wb), _f32)

    @functools.partial(
        pl.kernel,
        out_type=(osh, oshb),
        mesh=mesh,
        scratch_types=[pltpu.VMEM((mch,), jnp.int32),
                       pltpu.VMEM((mch,), jnp.int32),
                       pltpu.VMEM((mch, width), _f32),
                       pltpu.VMEM((mch, width), _f32),
                       pltpu.SemaphoreType.DMA,
                       pltpu.SemaphoreType.DMA],
        compiler_params=None if tiled else pltpu.CompilerParams(
            use_tc_tiling_on_sc=False),
    )
    def k(ta_h, tb_h, src_h, dst_h, ga_h, gb_h, idx_s, idx_d, row_a, row_b,
          sem, sem2):
        w = _wid()

        def do(cid):
            base = cid * mch
            c1 = pltpu.async_copy(src_h.at[pl.ds(base, mch)], idx_s, sem)
            c2 = pltpu.async_copy(dst_h.at[pl.ds(base, mch)], idx_d, sem2)
            c1.wait()
            c2.wait()
            ds_ = []
            for j in range(sub):
                sl = pl.ds(j * CHUNK, CHUNK)
                ds_.append(pltpu.async_copy(
                    ta_h.at[idx_s.at[sl]], row_a.at[sl], sem))
                ds_.append(pltpu.async_copy(
                    tb_h.at[idx_d.at[sl]], row_b.at[sl], sem2))
            for d in ds_:
                d.wait()
            w1 = pltpu.async_copy(row_a, ga_h.at[pl.ds(base, mch)], sem)
            w2 = pltpu.async_copy(row_b if wb == width
                                  else row_b.at[:, pl.ds(0, wb)],
                                  gb_h.at[pl.ds(base, mch)], sem2)
            w1.wait()
            w2.wait()

        _strided_chunks(nchunks, NW, w, do)

    return k(ta, tb, src, dst)


def sc_segmax(alpha1, dst, np_):
    e = dst.shape[0]
    mch = 3200
    nchunks = e // mch
    nseg = np_ // LANES
    rtiles = NW // HEADS
    mesh = plsc.VectorSubcoreMesh(**_MESH)

    @functools.partial(
        pl.kernel,
        out_type=jax.ShapeDtypeStruct((NW * np_,), _f32),
        mesh=mesh,
        scratch_types=[pltpu.VMEM((mch,), jnp.int32),
                       pltpu.VMEM((mch,), _f32),
                       pltpu.VMEM((np_,), _f32)],
        compiler_params=pltpu.CompilerParams(needs_layout_passes=False),
    )
    def k(al_h, dst_h, out_h, dbuf, abuf, maxv):
        w = _wid()
        h = w // rtiles
        r = w % rtiles
        neg = jnp.full((LANES,), NEG, _f32)

        def init(i, c):
            maxv[pl.ds(i * LANES, LANES)] = neg
            return c
        lax.fori_loop(0, nseg, init, 0)

        def do(cid):
            base = cid * mch
            pltpu.sync_copy(dst_h.at[pl.ds(base, mch)], dbuf)
            pltpu.sync_copy(al_h.at[pl.ds(h * e + base, mch)], abuf)

            def grp(g, c):
                d = dbuf[pl.ds(g * LANES, LANES)]
                a = abuf[pl.ds(g * LANES, LANES)]
                m = plsc.load_gather(maxv, [d])
                x = jnp.maximum(m, a)
                plsc.store_scatter(maxv, [d], x)
                rd = plsc.load_gather(maxv, [d])
                lost = rd < x

                def fix_cond(carry):
                    _, lost_ = carry
                    return jnp.any(lost_)

                def fix_body(carry):
                    rd_, lost_ = carry
                    plsc.store_scatter(maxv, [d], jnp.maximum(rd_, x),
                                       mask=lost_)
                    rd2 = plsc.load_gather(maxv, [d])
                    return rd2, rd2 < x

                lax.while_loop(fix_cond, fix_body, (rd, lost))
                return c

            lax.fori_loop(0, mch // LANES, grp, 0)

        _strided_chunks(nchunks, rtiles, r, do)
        pltpu.sync_copy(maxv, out_h.at[pl.ds(w * np_, np_)])

    return k(alpha1, dst)


def sc_exdenom(alpha1, amax1, dst, np_):
    e = dst.shape[0]
    mch = 3200
    nchunks = e // mch
    nseg = np_ // LANES
    rtiles = NW // HEADS
    mesh = plsc.VectorSubcoreMesh(**_MESH)

    @functools.partial(
        pl.kernel,
        out_type=(jax.ShapeDtypeStruct((HEADS * e,), _f32),
                  jax.ShapeDtypeStruct((NW * np_,), _f32)),
        mesh=mesh,
        scratch_types=[pltpu.VMEM((mch,), jnp.int32),
                       pltpu.VMEM((mch,), _f32),
                       pltpu.VMEM((mch,), _f32),
                       pltpu.VMEM((np_,), _f32),
                       pltpu.VMEM((np_,), _f32)],
        compiler_params=pltpu.CompilerParams(needs_layout_passes=False),
    )
    def k(al_h, amax_h, dst_h, ex_h, den_h,
          dbuf, abuf, exbuf, amaxv, denv):
        w = _wid()
        h = w // rtiles
        r = w % rtiles
        zero = jnp.zeros((LANES,), _f32)

        pltpu.sync_copy(amax_h.at[pl.ds(h * np_, np_)], amaxv)

        def init(i, c):
            denv[pl.ds(i * LANES, LANES)] = zero
            return c
        lax.fori_loop(0, nseg, init, 0)

        def do(cid):
            base = cid * mch
            pltpu.sync_copy(dst_h.at[pl.ds(base, mch)], dbuf)
            pltpu.sync_copy(al_h.at[pl.ds(h * e + base, mch)], abuf)

            def grp(g, c):
                d = dbuf[pl.ds(g * LANES, LANES)]
                a = abuf[pl.ds(g * LANES, LANES)]
                m = plsc.load_gather(amaxv, [d])
                ex = jnp.exp(a - m)
                exbuf[pl.ds(g * LANES, LANES)] = ex
                plsc.addupdate_scatter(denv, [d], ex)
                return c

            lax.fori_loop(0, mch // LANES, grp, 0)
            pltpu.sync_copy(exbuf, ex_h.at[pl.ds(h * e + base, mch)])

        _strided_chunks(nchunks, rtiles, r, do)
        pltpu.sync_copy(denv, den_h.at[pl.ds(w * np_, np_)])

    return k(alpha1, amax1, dst)


def sc_scatter(r2, dst, np_):
    e = dst.shape[0]
    sub = 5
    mch = CHUNK * sub
    nchunks = e // mch
    half = HID // 2
    rows_per_tile = np_ // NSUB
    zchunk = 64
    nz = rows_per_tile // zchunk
    mesh = plsc.VectorSubcoreMesh(**_MESH)

    @functools.partial(
        pl.kernel,
        out_type=jax.ShapeDtypeStruct((NCORES, np_, half), _f32),
        mesh=mesh,
        scratch_types=[pltpu.VMEM((sub, CHUNK), jnp.int32),
                       pltpu.VMEM((mch, half), _f32),
                       pltpu.VMEM((zchunk, half), _f32),
                       pltpu.VMEM_SHARED((np_, half), _f32),
                       pltpu.SemaphoreType.DMA,
                       pltpu.SemaphoreType.DMA],
        compiler_params=pltpu.CompilerParams(use_tc_tiling_on_sc=False),
    )
    def k(r_h, dst2_h, acc_h, idxb, rowb, zbuf, accum, sem, sem2):
        c = lax.axis_index("c")
        s = lax.axis_index("s")
        zero = jnp.zeros((LANES,), _f32)
        zrows = zchunk * half // LANES

        def zinit(i, cc):
            zbuf[i // (half // LANES), pl.ds((i % (half // LANES)) * LANES,
                                             LANES)] = zero
            return cc
        lax.fori_loop(0, zrows, zinit, 0)

        def zcopy(j, cc):
            pltpu.sync_copy(
                zbuf, accum.at[pl.ds(s * rows_per_tile + j * zchunk, zchunk)])
            return cc
        lax.fori_loop(0, nz, zcopy, 0)
        plsc.subcore_barrier()

        def do(cid):
            base = cid * mch
            c1 = pltpu.async_copy(dst2_h.at[pl.ds(cid * sub, sub)], idxb, sem)
            c2 = pltpu.async_copy(r_h.at[c, pl.ds(base, mch)], rowb, sem2)
            c1.wait()
            c2.wait()
            descs = []
            for j in range(sub):
                descs.append(pltpu.async_copy(
                    rowb.at[pl.ds(j * CHUNK, CHUNK)],
                    accum.at[idxb.at[j]], sem, add=True))
            for d in descs:
                d.wait()

        _strided_chunks(nchunks, NSUB, s, do)
        plsc.subcore_barrier()
        pltpu.sync_copy(accum.at[pl.ds(s * rows_per_tile, rows_per_tile)],
                        acc_h.at[c, pl.ds(s * rows_per_tile, rows_per_tile)])

    return k(r2, dst)


def _conv(x, src, dst, ep, p, np_):
    kv, qq, s = tc_proj4(x, p)
    kvg, qg = sc_gather2(kv, qq, src, dst)
    alpha, vpe = tc_alpha(qg, kvg, ep)
    e = src.shape[0]
    maxpart = sc_segmax(alpha.reshape(HEADS * e), dst, np_)
    amax = tc_reduce(maxpart.reshape(NW // HEADS, HEADS, np_), 'max')
    ex1, denpart = sc_exdenom(alpha.reshape(HEADS * e),
                              amax.reshape(HEADS * np_), dst, np_)
    den = tc_reduce(denpart.reshape(NW // HEADS, HEADS, np_), 'sum')
    r2 = tc_rpass(ex1.reshape(HEADS, e), vpe)
    acc = sc_scatter(r2, dst.reshape(e // CHUNK, CHUNK), np_)
    return tc_finalize(acc, den, s)


def _graph_net(x, src, dst, edge_attr, gp, np_):
    for p in gp['convs']:
        ep = tc_linear(edge_attr, p['We_pad'], p['be'], blk=2000)
        x = _conv(x, src, dst, ep, p, np_)
    return tc_mlp(x, gp['W1'], gp['b1'], gp['W2'], gp['b2'])


def kernel(f, x_t, edge_index, batch, params):
    n = f.shape[0]
    np_ = _pad_nodes(n)
    src, dst = edge_index[0], edge_index[1]

    xp = jnp.pad(x_t, ((0, np_ - n), (0, 16 - x_t.shape[1])))
    fp = jnp.pad(f, ((0, np_ - n), (0, 0)))

    def prep(gp):
        convs = []
        for p in gp['convs']:
            p2 = dict(p)
            p2['We_pad'] = jnp.pad(p['We'],
                                   ((0, EDGE_PAD - p['We'].shape[0]), (0, 0)))
            convs.append(p2)
        return {'convs': convs, 'W1': gp['W1'], 'b1': gp['b1'],
                'W2': gp['W2'], 'b2': gp['b2']}

    cond_p = prep(params['cond'])
    b_p = prep(params['b_net'])
    eta_p = prep(params['eta_net'])

    sg, dg = sc_gather2(xp, xp, src, dst, width=16, tiled=False)
    edge_attr = tc_edge_attr(sg, dg)

    h = tc_linear(fp, params['W_in'], params['b_in'])
    f_cond = _graph_net(h, src, dst, edge_attr, cond_p, np_)
    b = _graph_net(f_cond, src, dst, edge_attr, b_p, np_)
    eta = _graph_net(f_cond, src, dst, edge_attr, eta_p, np_)
    return f_cond[:n], b[:n], eta[:n]


# speedup vs baseline: 1.0080x; 1.0080x over previous
import functools

import jax
import jax.numpy as jnp
from jax import lax
from jax.experimental import pallas as pl
from jax.experimental.pallas import tpu as pltpu
from jax.experimental.pallas import tpu_sc as plsc

HEADS = 4
CH = 16
HID = 64
NUM_BASIS = 16
MAX_RADIUS = 5.0
EDGE_PAD = 32
WIDE = 128

NCORES = 2
NSUB = 16
NW = NCORES * NSUB
LANES = 16
CHUNK = 128
NEG = -1e30

_f32 = jnp.float32


def _mm(a, b, dims):
    return lax.dot_general(a, b, (dims, ((), ())),
                           preferred_element_type=jnp.float32)


def _pad_nodes(n):
    m = 128 * NSUB
    return ((n + m - 1) // m) * m


def _lin_body(relu, x_ref, w_ref, b_ref, o_ref):
    y = _mm(x_ref[...], w_ref[...], ((1,), (0,))) + b_ref[...]
    o_ref[...] = jnp.maximum(y, 0.0) if relu else y


def tc_linear(x, w, b, blk=2048, relu=False):
    n, din = x.shape
    dout = w.shape[1]
    return pl.pallas_call(
        functools.partial(_lin_body, relu),
        grid=(n // blk,),
        in_specs=[pl.BlockSpec((blk, din), lambda i: (i, 0)),
                  pl.BlockSpec((din, dout), lambda i: (0, 0)),
                  pl.BlockSpec((1, dout), lambda i: (0, 0))],
        out_specs=pl.BlockSpec((blk, dout), lambda i: (i, 0)),
        out_shape=jax.ShapeDtypeStruct((n, dout), _f32),
    )(x, w, b.reshape(1, -1))


def _mlp_body(x_ref, w1_ref, b1_ref, w2_ref, b2_ref, o_ref):
    h = jnp.maximum(_mm(x_ref[...], w1_ref[...], ((1,), (0,))) + b1_ref[...], 0.0)
    o_ref[...] = _mm(h, w2_ref[...], ((1,), (0,))) + b2_ref[...]


def tc_mlp(x, w1, b1, w2, b2, blk=2048):
    n, din = x.shape
    dh = w1.shape[1]
    dout = w2.shape[1]
    return pl.pallas_call(
        _mlp_body,
        grid=(n // blk,),
        in_specs=[pl.BlockSpec((blk, din), lambda i: (i, 0)),
                  pl.BlockSpec((din, dh), lambda i: (0, 0)),
                  pl.BlockSpec((1, dh), lambda i: (0, 0)),
                  pl.BlockSpec((dh, dout), lambda i: (0, 0)),
                  pl.BlockSpec((1, dout), lambda i: (0, 0))],
        out_specs=pl.BlockSpec((blk, dout), lambda i: (i, 0)),
        out_shape=jax.ShapeDtypeStruct((n, dout), _f32),
    )(x, w1, b1.reshape(1, -1), w2, b2.reshape(1, -1))


def _proj4_body(x_ref, wq, bq, wk, bk, wv, bv, ws, bs, kv_o, qq_o, s_o):
    x = x_ref[...]
    q = _mm(x, wq[...], ((1,), (0,))) + bq[...]
    k = _mm(x, wk[...], ((1,), (0,))) + bk[...]
    v = _mm(x, wv[...], ((1,), (0,))) + bv[...]
    kv_o[...] = jnp.concatenate([k, v], axis=1)
    qq_o[...] = jnp.concatenate([q, q], axis=1)
    s_o[...] = _mm(x, ws[...], ((1,), (0,))) + bs[...]


def tc_proj4(x, p, blk=2048):
    n, din = x.shape
    wspec = pl.BlockSpec((din, HID), lambda i: (0, 0))
    bspec = pl.BlockSpec((1, HID), lambda i: (0, 0))
    return pl.pallas_call(
        _proj4_body,
        grid=(n // blk,),
        in_specs=[pl.BlockSpec((blk, din), lambda i: (i, 0)),
                  wspec, bspec, wspec, bspec, wspec, bspec, wspec, bspec],
        out_specs=[pl.BlockSpec((blk, WIDE), lambda i: (i, 0)),
                   pl.BlockSpec((blk, WIDE), lambda i: (i, 0)),
                   pl.BlockSpec((blk, HID), lambda i: (i, 0))],
        out_shape=[jax.ShapeDtypeStruct((n, WIDE), _f32),
                   jax.ShapeDtypeStruct((n, WIDE), _f32),
                   jax.ShapeDtypeStruct((n, HID), _f32)],
    )(x, p['Wq'], p['bq'].reshape(1, -1), p['Wk'], p['bk'].reshape(1, -1),
      p['Wv'], p['bv'].reshape(1, -1), p['Ws'], p['bs'].reshape(1, -1))


def _edge_attr_body(sg_ref, dg_ref, o_ref):
    ev = sg_ref[:, :XP3] - dg_ref[:, :XP3]
    len2 = jnp.sum(ev * ev, axis=1, keepdims=True)
    elen = jnp.sqrt(len2)
    t = elen * _f32((NUM_BASIS + 1) / MAX_RADIUS)
    j = lax.broadcasted_iota(jnp.int32, (1, NUM_BASIS), 1).astype(_f32) + 1.0
    diff = t - j
    emb = jnp.exp(-(diff * diff)) * _f32(NUM_BASIS ** 0.5 / 1.12)
    unit = ev[:, :3] / (elen + 1e-12)
    b = sg_ref.shape[0]
    ones = jnp.ones((b, 1), _f32)
    zeros = jnp.zeros((b, EDGE_PAD - NUM_BASIS - 4), _f32)
    o_ref[...] = jnp.concatenate(
        [emb, ones, _f32(3.0 ** 0.5) * unit, zeros], axis=1)


XP3 = 8


def tc_edge_attr(sg, dg, blk=2000):
    e = sg.shape[0]
    return pl.pallas_call(
        _edge_attr_body,
        grid=(e // blk,),
        in_specs=[pl.BlockSpec((blk, 16), lambda i: (i, 0)),
                  pl.BlockSpec((blk, 16), lambda i: (i, 0))],
        out_specs=pl.BlockSpec((blk, EDGE_PAD), lambda i: (i, 0)),
        out_shape=jax.ShapeDtypeStruct((e, EDGE_PAD), _f32),
    )(sg, dg)


def _alpha_body(qq_ref, kv_ref, ep_ref, a_o, vpe_o):
    ep = ep_ref[...]
    qk = qq_ref[:, :HID] * (kv_ref[:, :HID] + ep)
    r = lax.broadcasted_iota(jnp.int32, (HID, HEADS), 0)
    c = lax.broadcasted_iota(jnp.int32, (HID, HEADS), 1)
    hsel = jnp.where(r // CH == c, _f32(1.0 / (CH ** 0.5)), 0.0)
    a_o[...] = _mm(hsel, qk, ((0,), (1,)))
    vpe_o[...] = kv_ref[:, HID:] + ep


def tc_alpha(qq, kv, ep, blk=6400):
    e = qq.shape[0]
    return pl.pallas_call(
        _alpha_body,
        grid=(e // blk,),
        in_specs=[pl.BlockSpec((blk, WIDE), lambda i: (i, 0)),
                  pl.BlockSpec((blk, WIDE), lambda i: (i, 0)),
                  pl.BlockSpec((blk, HID), lambda i: (i, 0))],
        out_specs=[pl.BlockSpec((HEADS, blk), lambda i: (0, i)),
                   pl.BlockSpec((blk, HID), lambda i: (i, 0))],
        out_shape=[jax.ShapeDtypeStruct((HEADS, e), _f32),
                   jax.ShapeDtypeStruct((e, HID), _f32)],
    )(qq, kv, ep)


def _maxreduce_body(p_ref, o_ref):
    m = jnp.max(p_ref[...], axis=0)
    o_ref[...] = jnp.where(m < _f32(-1e29), 0.0, m)


def _sumreduce_body(p_ref, o_ref):
    o_ref[...] = jnp.sum(p_ref[...], axis=0)


def tc_reduce(parts, kind):
    r, h, n = parts.shape
    body = _maxreduce_body if kind == 'max' else _sumreduce_body
    return pl.pallas_call(
        body,
        grid=(1,),
        in_specs=[pl.BlockSpec((r, h, n), lambda i: (0, 0, 0))],
        out_specs=pl.BlockSpec((h, n), lambda i: (0, 0)),
        out_shape=jax.ShapeDtypeStruct((h, n), _f32),
    )(parts)


def _b4():
    r = lax.broadcasted_iota(jnp.int32, (HEADS, HID), 0)
    c = lax.broadcasted_iota(jnp.int32, (HEADS, HID), 1)
    return jnp.where(c // CH == r, _f32(1.0), 0.0)


def _rpass_body(ex_ref, vpe_ref, o_ref):
    exb = _mm(ex_ref[...], _b4(), ((0,), (0,)))
    rfull = exb * vpe_ref[...]
    o_ref[0] = rfull[:, :HID // 2]
    o_ref[1] = rfull[:, HID // 2:]


def tc_rpass(ex, vpe, blk=6400):
    e = vpe.shape[0]
    return pl.pallas_call(
        _rpass_body,
        grid=(e // blk,),
        in_specs=[pl.BlockSpec((HEADS, blk), lambda i: (0, i)),
                  pl.BlockSpec((blk, HID), lambda i: (i, 0))],
        out_specs=pl.BlockSpec((2, blk, HID // 2), lambda i: (0, i, 0)),
        out_shape=jax.ShapeDtypeStruct((2, e, HID // 2), _f32),
    )(ex, vpe)


def _finalize_body(blk, acc_ref, den_ref, s_ref, o_ref):
    i = pl.program_id(0)
    acc = jnp.concatenate([acc_ref[0], acc_ref[1]], axis=1)
    den = den_ref[:, pl.ds(i * blk, blk)]
    db = _mm(den, _b4(), ((0,), (0,)))
    o_ref[...] = jnp.maximum(acc / (db + _f32(1e-16)) + s_ref[...], 0.0)


def tc_finalize(acc, den, s, blk=2048):
    n = s.shape[0]
    np_ = den.shape[1]
    return pl.pallas_call(
        functools.partial(_finalize_body, blk),
        grid=(n // blk,),
        in_specs=[pl.BlockSpec((2, blk, HID // 2), lambda i: (0, i, 0)),
                  pl.BlockSpec((HEADS, np_), lambda i: (0, 0)),
                  pl.BlockSpec((blk, HID), lambda i: (i, 0))],
        out_specs=pl.BlockSpec((blk, HID), lambda i: (i, 0)),
        out_shape=jax.ShapeDtypeStruct((n, HID), _f32),
    )(acc, den, s)


_MESH = dict(core_axis_name="c", subcore_axis_name="s")


def _wid():
    return lax.axis_index("c") * NSUB + lax.axis_index("s")


def _strided_chunks(nchunks, stride, offset, body):
    total = (nchunks - offset + stride - 1) // stride

    def outer(j, carry):
        body(offset + j * stride)
        return carry

    lax.fori_loop(0, total, outer, 0)


def sc_gather2(ta, tb, src, dst, width=WIDE, tiled=True, sub=2, bw_b=None):
    e = src.shape[0]
    mch = CHUNK * sub
    nchunks = e // mch
    wb = width if bw_b is None else bw_b
    mesh = plsc.VectorSubcoreMesh(**_MESH)
    osh = jax.ShapeDtypeStruct((e, width), _f32)
    oshb = jax.ShapeDtypeStruct((e, wb), _f32)

    @functools.partial(
        pl.kernel,
        out_type=(osh, oshb),
        mesh=mesh,
        scratch_types=[pltpu.VMEM((mch,), jnp.int32),
                       pltpu.VMEM((mch,), jnp.int32),
                       pltpu.VMEM((mch, width), _f32),
                       pltpu.VMEM((mch, width), _f32),
                       pltpu.SemaphoreType.DMA,
                       pltpu.SemaphoreType.DMA],
        compiler_params=None if tiled else pltpu.CompilerParams(
            use_tc_tiling_on_sc=False),
    )
    def k(ta_h, tb_h, src_h, dst_h, ga_h, gb_h, idx_s, idx_d, row_a, row_b,
          sem, sem2):
        w = _wid()

        def do(cid):
            base = cid * mch
            c1 = pltpu.async_copy(src_h.at[pl.ds(base, mch)], idx_s, sem)
            c2 = pltpu.async_copy(dst_h.at[pl.ds(base, mch)], idx_d, sem2)
            c1.wait()
            c2.wait()
            ds_ = []
            for j in range(sub):
                sl = pl.ds(j * CHUNK, CHUNK)
                ds_.append(pltpu.async_copy(
                    ta_h.at[idx_s.at[sl]], row_a.at[sl], sem))
                ds_.append(pltpu.async_copy(
                    tb_h.at[idx_d.at[sl]], row_b.at[sl], sem2))
            for d in ds_:
                d.wait()
            w1 = pltpu.async_copy(row_a, ga_h.at[pl.ds(base, mch)], sem)
            w2 = pltpu.async_copy(row_b if wb == width
                                  else row_b.at[:, pl.ds(0, wb)],
                                  gb_h.at[pl.ds(base, mch)], sem2)
            w1.wait()
            w2.wait()

        _strided_chunks(nchunks, NW, w, do)

    return k(ta, tb, src, dst)


def sc_segmax(alpha1, dst, np_):
    e = dst.shape[0]
    mch = 6400
    nchunks = e // mch
    nseg = np_ // LANES
    rtiles = NW // HEADS
    mesh = plsc.VectorSubcoreMesh(**_MESH)

    @functools.partial(
        pl.kernel,
        out_type=jax.ShapeDtypeStruct((NW * np_,), _f32),
        mesh=mesh,
        scratch_types=[pltpu.VMEM((mch,), jnp.int32),
                       pltpu.VMEM((mch,), _f32),
                       pltpu.VMEM((np_,), _f32)],
        compiler_params=pltpu.CompilerParams(needs_layout_passes=False),
    )
    def k(al_h, dst_h, out_h, dbuf, abuf, maxv):
        w = _wid()
        h = w // rtiles
        r = w % rtiles
        neg = jnp.full((LANES,), NEG, _f32)

        def init(i, c):
            maxv[pl.ds(i * LANES, LANES)] = neg
            return c
        lax.fori_loop(0, nseg, init, 0)

        def do(cid):
            base = cid * mch
            pltpu.sync_copy(dst_h.at[pl.ds(base, mch)], dbuf)
            pltpu.sync_copy(al_h.at[pl.ds(h * e + base, mch)], abuf)

            def grp(g, c):
                d = dbuf[pl.ds(g * LANES, LANES)]
                a = abuf[pl.ds(g * LANES, LANES)]
                m = plsc.load_gather(maxv, [d])
                x = jnp.maximum(m, a)
                plsc.store_scatter(maxv, [d], x)
                rd = plsc.load_gather(maxv, [d])
                lost = rd < x

                def fix_cond(carry):
                    _, lost_ = carry
                    return jnp.any(lost_)

                def fix_body(carry):
                    rd_, lost_ = carry
                    plsc.store_scatter(maxv, [d], jnp.maximum(rd_, x),
                                       mask=lost_)
                    rd2 = plsc.load_gather(maxv, [d])
                    return rd2, rd2 < x

                lax.while_loop(fix_cond, fix_body, (rd, lost))
                return c

            lax.fori_loop(0, mch // LANES, grp, 0)

        _strided_chunks(nchunks, rtiles, r, do)
        pltpu.sync_copy(maxv, out_h.at[pl.ds(w * np_, np_)])

    return k(alpha1, dst)


def sc_exdenom(alpha1, amax1, dst, np_):
    e = dst.shape[0]
    mch = 6400
    nchunks = e // mch
    nseg = np_ // LANES
    rtiles = NW // HEADS
    mesh = plsc.VectorSubcoreMesh(**_MESH)

    @functools.partial(
        pl.kernel,
        out_type=(jax.ShapeDtypeStruct((HEADS * e,), _f32),
                  jax.ShapeDtypeStruct((NW * np_,), _f32)),
        mesh=mesh,
        scratch_types=[pltpu.VMEM((mch,), jnp.int32),
                       pltpu.VMEM((mch,), _f32),
                       pltpu.VMEM((mch,), _f32),
                       pltpu.VMEM((np_,), _f32),
                       pltpu.VMEM((np_,), _f32)],
        compiler_params=pltpu.CompilerParams(needs_layout_passes=False),
    )
    def k(al_h, amax_h, dst_h, ex_h, den_h,
          dbuf, abuf, exbuf, amaxv, denv):
        w = _wid()
        h = w // rtiles
        r = w % rtiles
        zero = jnp.zeros((LANES,), _f32)

        pltpu.sync_copy(amax_h.at[pl.ds(h * np_, np_)], amaxv)

        def init(i, c):
            denv[pl.ds(i * LANES, LANES)] = zero
            return c
        lax.fori_loop(0, nseg, init, 0)

        def do(cid):
            base = cid * mch
            pltpu.sync_copy(dst_h.at[pl.ds(base, mch)], dbuf)
            pltpu.sync_copy(al_h.at[pl.ds(h * e + base, mch)], abuf)

            def grp(g, c):
                d = dbuf[pl.ds(g * LANES, LANES)]
                a = abuf[pl.ds(g * LANES, LANES)]
                m = plsc.load_gather(amaxv, [d])
                ex = jnp.exp(a - m)
                exbuf[pl.ds(g * LANES, LANES)] = ex
                plsc.addupdate_scatter(denv, [d], ex)
                return c

            lax.fori_loop(0, mch // LANES, grp, 0)
            pltpu.sync_copy(exbuf, ex_h.at[pl.ds(h * e + base, mch)])

        _strided_chunks(nchunks, rtiles, r, do)
        pltpu.sync_copy(denv, den_h.at[pl.ds(w * np_, np_)])

    return k(alpha1, amax1, dst)


def sc_scatter(r2, dst, np_):
    e = dst.shape[0]
    sub = 5
    mch = CHUNK * sub
    nchunks = e // mch
    half = HID // 2
    rows_per_tile = np_ // NSUB
    zchunk = 64
    nz = rows_per_tile // zchunk
    mesh = plsc.VectorSubcoreMesh(**_MESH)

    @functools.partial(
        pl.kernel,
        out_type=jax.ShapeDtypeStruct((NCORES, np_, half), _f32),
        mesh=mesh,
        scratch_types=[pltpu.VMEM((sub, CHUNK), jnp.int32),
                       pltpu.VMEM((mch, half), _f32),
                       pltpu.VMEM((zchunk, half), _f32),
                       pltpu.VMEM_SHARED((np_, half), _f32),
                       pltpu.SemaphoreType.DMA,
                       pltpu.SemaphoreType.DMA],
        compiler_params=pltpu.CompilerParams(use_tc_tiling_on_sc=False),
    )
    def k(r_h, dst2_h, acc_h, idxb, rowb, zbuf, accum, sem, sem2):
        c = lax.axis_index("c")
        s = lax.axis_index("s")
        zero = jnp.zeros((LANES,), _f32)
        zrows = zchunk * half // LANES

        def zinit(i, cc):
            zbuf[i // (half // LANES), pl.ds((i % (half // LANES)) * LANES,
                                             LANES)] = zero
            return cc
        lax.fori_loop(0, zrows, zinit, 0)

        def zcopy(j, cc):
            pltpu.sync_copy(
                zbuf, accum.at[pl.ds(s * rows_per_tile + j * zchunk, zchunk)])
            return cc
        lax.fori_loop(0, nz, zcopy, 0)
        plsc.subcore_barrier()

        def do(cid):
            base = cid * mch
            c1 = pltpu.async_copy(dst2_h.at[pl.ds(cid * sub, sub)], idxb, sem)
            c2 = pltpu.async_copy(r_h.at[c, pl.ds(base, mch)], rowb, sem2)
            c1.wait()
            c2.wait()
            descs = []
            for j in range(sub):
                descs.append(pltpu.async_copy(
                    rowb.at[pl.ds(j * CHUNK, CHUNK)],
                    accum.at[idxb.at[j]], sem, add=True))
            for d in descs:
                d.wait()

        _strided_chunks(nchunks, NSUB, s, do)
        plsc.subcore_barrier()
        pltpu.sync_copy(accum.at[pl.ds(s * rows_per_tile, rows_per_tile)],
                        acc_h.at[c, pl.ds(s * rows_per_tile, rows_per_tile)])

    return k(r2, dst)


def _conv(x, src, dst, ep, p, np_):
    kv, qq, s = tc_proj4(x, p)
    kvg, qg = sc_gather2(kv, qq, src, dst)
    alpha, vpe = tc_alpha(qg, kvg, ep)
    e = src.shape[0]
    maxpart = sc_segmax(alpha.reshape(HEADS * e), dst, np_)
    amax = tc_reduce(maxpart.reshape(NW // HEADS, HEADS, np_), 'max')
    ex1, denpart = sc_exdenom(alpha.reshape(HEADS * e),
                              amax.reshape(HEADS * np_), dst, np_)
    den = tc_reduce(denpart.reshape(NW // HEADS, HEADS, np_), 'sum')
    r2 = tc_rpass(ex1.reshape(HEADS, e), vpe)
    acc = sc_scatter(r2, dst.reshape(e // CHUNK, CHUNK), np_)
    return tc_finalize(acc, den, s)


def _graph_net(x, src, dst, edge_attr, gp, np_):
    for p in gp['convs']:
        ep = tc_linear(edge_attr, p['We_pad'], p['be'], blk=2000)
        x = _conv(x, src, dst, ep, p, np_)
    return tc_mlp(x, gp['W1'], gp['b1'], gp['W2'], gp['b2'])


def kernel(f, x_t, edge_index, batch, params):
    n = f.shape[0]
    np_ = _pad_nodes(n)
    src, dst = edge_index[0], edge_index[1]

    xp = jnp.pad(x_t, ((0, np_ - n), (0, 16 - x_t.shape[1])))
    fp = jnp.pad(f, ((0, np_ - n), (0, 0)))

    def prep(gp):
        convs = []
        for p in gp['convs']:
            p2 = dict(p)
            p2['We_pad'] = jnp.pad(p['We'],
                                   ((0, EDGE_PAD - p['We'].shape[0]), (0, 0)))
            convs.append(p2)
        return {'convs': convs, 'W1': gp['W1'], 'b1': gp['b1'],
                'W2': gp['W2'], 'b2': gp['b2']}

    cond_p = prep(params['cond'])
    b_p = prep(params['b_net'])
    eta_p = prep(params['eta_net'])

    sg, dg = sc_gather2(xp, xp, src, dst, width=16, tiled=False)
    edge_attr = tc_edge_attr(sg, dg)

    h = tc_linear(fp, params['W_in'], params['b_in'])
    f_cond = _graph_net(h, src, dst, edge_attr, cond_p, np_)
    b = _graph_net(f_cond, src, dst, edge_attr, b_p, np_)
    eta = _graph_net(f_cond, src, dst, edge_attr, eta_p, np_)
    return f_cond[:n], b[:n], eta[:n]
